# baseline (speedup 1.0000x reference)
import dataclasses
import functools

import jax
import jax.numpy as jnp
from jax import lax
from jax.experimental import pallas as pl
from jax.experimental.pallas import tpu as pltpu
from jax.experimental.pallas import tpu_sc as plsc

F32 = jnp.float32
I32 = jnp.int32
DH = 128
NC, NS = 2, 16
NW = NC * NS
CH = 128

N_SIZES = (50000, 50000, 10000, 10000)
T_SRCS = (0, 1, 2, 3)
T_TGTS = (1, 0, 3, 2)
N_LAYERS = 5

R_BIG, R_SMALL = 8448, 5120
N_PADS = (6 * R_BIG, 6 * R_BIG, 2 * R_SMALL, 2 * R_SMALL)
R_SIZES = (R_BIG, R_BIG, R_SMALL, R_SMALL)
NBS = (6, 6, 2, 2)
E_PADS = (200704, 200704, 100352, 100352)
SEGS = (6272, 6272, 3200, 3200)


def _mm_body(x_ref, w_ref, b_ref, o_ref, *, in_relu):
    x = x_ref[...]
    if in_relu:
        x = jnp.maximum(x, 0.0)
    acc = jnp.dot(x, w_ref[...], preferred_element_type=F32) + b_ref[...]
    o_ref[...] = jnp.maximum(acc, 0.0)


def _msg_mm(x, w, b, in_relu):
    n, k = x.shape
    blk = 512
    return pl.pallas_call(
        functools.partial(_mm_body, in_relu=in_relu),
        grid=(n // blk,),
        in_specs=[
            pl.BlockSpec((blk, k), lambda i: (i, 0)),
            pl.BlockSpec((k, DH), lambda i: (0, 0)),
            pl.BlockSpec((1, DH), lambda i: (0, 0)),
        ],
        out_specs=pl.BlockSpec((blk, DH), lambda i: (i, 0)),
        out_shape=jax.ShapeDtypeStruct((n, DH), F32),
    )(x, w, b.reshape(1, DH))


def _enc2_body(xa_ref, xb_ref, wa_ref, wb_ref, b_ref, o_ref):
    acc = jnp.dot(xa_ref[...], wa_ref[...], preferred_element_type=F32)
    acc += jnp.dot(xb_ref[...], wb_ref[...], preferred_element_type=F32)
    o_ref[...] = acc + b_ref[...]


def _enc2(xa, xb, wa, wb, b):
    n, ka = xa.shape
    kb = xb.shape[1]
    blk = 512
    return pl.pallas_call(
        _enc2_body,
        grid=(n // blk,),
        in_specs=[
            pl.BlockSpec((blk, ka), lambda i: (i, 0)),
            pl.BlockSpec((blk, kb), lambda i: (i, 0)),
            pl.BlockSpec((ka, DH), lambda i: (0, 0)),
            pl.BlockSpec((kb, DH), lambda i: (0, 0)),
            pl.BlockSpec((1, DH), lambda i: (0, 0)),
        ],
        out_specs=pl.BlockSpec((blk, DH), lambda i: (i, 0)),
        out_shape=jax.ShapeDtypeStruct((n, DH), F32),
    )(xa, xb, wa, wb, b.reshape(1, DH))


def _emb_body(x_ref, e_ref, o_ref):
    o_ref[...] = x_ref[...] * e_ref[...]


def _enc_emb(x, emb):
    n = x.shape[0]
    blk = 512
    return pl.pallas_call(
        _emb_body,
        grid=(n // blk,),
        in_specs=[
            pl.BlockSpec((blk, 1), lambda i: (i, 0)),
            pl.BlockSpec((1, DH), lambda i: (0, 0)),
        ],
        out_specs=pl.BlockSpec((blk, DH), lambda i: (i, 0)),
        out_shape=jax.ShapeDtypeStruct((n, DH), F32),
    )(x, emb.reshape(1, DH))


def _relu_body(x_ref, o_ref):
    o_ref[...] = jnp.maximum(x_ref[...], 0.0)


def _relu_slice(x, n_out):
    blk = 400
    return pl.pallas_call(
        _relu_body,
        grid=(n_out // blk,),
        in_specs=[pl.BlockSpec((blk, DH), lambda i: (i, 0))],
        out_specs=pl.BlockSpec((blk, DH), lambda i: (i, 0)),
        out_shape=jax.ShapeDtypeStruct((n_out, DH), F32),
    )(x)


def _sc_mesh():
    return plsc.VectorSubcoreMesh(core_axis_name="c", subcore_axis_name="s")


def _sc_params():
    cp = pltpu.CompilerParams()
    if "needs_layout_passes" in pltpu.CompilerParams.__dataclass_fields__:
        cp = dataclasses.replace(cp, needs_layout_passes=False)
    return cp


@functools.lru_cache(maxsize=None)
def _make_sc_bucket(e_pad, nb, r, seg):
    per_w = e_pad // NW
    full_chunks = per_w // CH
    tail16 = (per_w - full_chunks * CH) // 16
    cap = seg // CH
    slot = 3 * seg

    @functools.partial(
        pl.kernel,
        out_type=(
            jax.ShapeDtypeStruct(((NW * nb * cap + 2) * 3 * CH,), I32),
            jax.ShapeDtypeStruct((NW * 16,), I32),
        ),
        mesh=_sc_mesh(),
        scratch_types=[
            pltpu.VMEM((CH,), I32),
            pltpu.VMEM((CH,), I32),
            pltpu.VMEM((CH,), F32),
            pltpu.VMEM((nb * slot,), I32),
            pltpu.VMEM((16,), I32),
        ],
        compiler_params=_sc_params(),
    )
    def bucket_k(src, tgt, w, oseg, ocnt, srcb, tgtb, wb, stg, scnt):
        c = lax.axis_index("c")
        s = lax.axis_index("s")
        wid = c * NS + s
        ebase = wid * per_w
        lanes = lax.iota(I32, 16)

        def do_chunk(e0, n16, pos):
            n = n16 * 16
            pltpu.sync_copy(src.at[pl.ds(e0, n)], srcb.at[pl.ds(0, n)])
            pltpu.sync_copy(tgt.at[pl.ds(e0, n)], tgtb.at[pl.ds(0, n)])
            pltpu.sync_copy(w.at[pl.ds(e0, n)], wb.at[pl.ds(0, n)])
            for gg in range(n16):
                sl = pl.ds(gg * 16, 16)
                tv = tgtb[sl]
                sv = srcb[sl]
                wv = plsc.bitcast(wb[sl], I32)
                bk = jnp.where(tv >= r, 1, 0)
                for m in range(2, nb):
                    bk = bk + jnp.where(tv >= m * r, 1, 0)
                p = jnp.zeros((16,), I32)
                for b in range(nb):
                    mb = bk == b
                    pc = plsc.cumsum(jnp.where(mb, 1, 0))
                    p = jnp.where(mb, b * seg + pos[b] + pc - 1, p)
                    pos = pos + jnp.where(lanes == b, pc[15], 0)
                blk = lax.shift_right_logical(p, 7)
                lane = lax.bitwise_and(p, 127)
                ps = blk * (3 * CH) + lane
                plsc.store_scatter(stg, [ps], sv)
                plsc.store_scatter(stg, [ps + CH], tv)
                plsc.store_scatter(stg, [ps + 2 * CH], wv)
            return pos

        def body(g, pos):
            return do_chunk(ebase + g * CH, CH // 16, pos)

        pos = lax.fori_loop(0, full_chunks, body, jnp.zeros((16,), I32))
        if tail16:
            pos = do_chunk(ebase + full_chunks * CH, tail16, pos)

        scnt[pl.ds(0, 16)] = pos
        for b in range(nb):
            o0 = (wid * nb + b) * slot
            pltpu.sync_copy(stg.at[pl.ds(b * slot, slot)],
                            oseg.at[pl.ds(o0, slot)])
        pltpu.sync_copy(scnt, ocnt.at[pl.ds(wid * 16, 16)])

    return bucket_k


@functools.lru_cache(maxsize=None)
def _make_sc_edge(n_tab, n_out, r, nb, seg):
    n_pass = nb // 2
    cap = seg // CH
    acc_rows = r + 16
    zone = acc_rows // NS
    out_ps = r // NS
    zsizes = [32] * (zone // 32) + ([zone % 32] if zone % 32 else [])

    @functools.partial(
        pl.kernel,
        out_type=jax.ShapeDtypeStruct((n_out, DH), F32),
        mesh=_sc_mesh(),
        scratch_types=[
            pltpu.VMEM((9, CH), I32),
            pltpu.VMEM((3, CH), I32),
            pltpu.VMEM((3, CH, DH), F32),
            pltpu.VMEM((32, DH), F32),
            pltpu.VMEM((NW * 16,), I32),
            pltpu.VMEM_SHARED((acc_rows, DH), F32),
            pltpu.SemaphoreType.DMA,
            pltpu.SemaphoreType.DMA,
            pltpu.SemaphoreType.DMA,
            pltpu.SemaphoreType.DMA,
            pltpu.SemaphoreType.DMA,
            pltpu.SemaphoreType.DMA,
        ],
        compiler_params=_sc_params(),
    )
    def edge_k(msg, eseg, cnts, out,
               meta, lclb, rows, zbuf, cntb, acc,
               gs0, gs1, gs2, ss0, ss1, ss2):
        gsems = (gs0, gs1, gs2)
        ssems = (ss0, ss1, ss2)
        c = lax.axis_index("c")
        s = lax.axis_index("s")
        lanes = lax.iota(I32, 16)
        pltpu.sync_copy(cnts, cntb)

        z16 = jnp.zeros((16,), F32)

        @pl.loop(0, 32)
        def _(rr):
            for cg in range(DH // 16):
                zbuf[rr, pl.ds(cg * 16, 16)] = z16

        def gather_desc(k):
            return pltpu.make_async_copy(
                msg.at[meta.at[3 * k]], rows.at[k], gsems[k])

        def scatter_desc(k):
            return pltpu.make_async_copy(
                rows.at[k], acc.at[lclb.at[k]], ssems[k])

        for p in range(n_pass):
            idx = 2 * p + c
            base = idx * r

            z0 = s * zone
            off = 0
            for ncopy in zsizes:
                pltpu.sync_copy(zbuf.at[pl.ds(0, ncopy)],
                                acc.at[pl.ds(z0 + off, ncopy)])
                off += ncopy
            plsc.subcore_barrier()

            for segi in range(2):
                sgm = 2 * s + segi
                cv = cntb[pl.ds(sgm * 16, 16)]
                count = jnp.sum(jnp.where(lanes == idx, cv, 0))
                nchunks = lax.div(count + (CH - 1), CH)
                t3 = lax.div(nchunks + 2, 3) * 3
                base_ci = (sgm * nb + idx) * cap

                def stage_a(q, k):
                    pltpu.sync_copy(eseg.at[base_ci + q],
                                    meta.at[pl.ds(3 * k, 3)])
                    for gg in range(CH // 16):
                        sl = pl.ds(gg * 16, 16)
                        sv = meta[3 * k, sl]
                        meta[3 * k, sl] = jnp.minimum(
                            jnp.maximum(sv, 0), n_tab - 1)
                        eid = q * CH + gg * 16 + lanes
                        lcl = meta[3 * k + 1, sl] - base
                        inr = (eid < count) & (lcl >= 0) & (lcl < r)
                        wv = plsc.bitcast(meta[3 * k + 2, sl], F32)
                        meta[3 * k + 2, sl] = plsc.bitcast(
                            jnp.where(inr, wv, 0.0), I32)
                        lclb[k, sl] = jnp.where(inr, lcl, r)
                    gather_desc(k).start()

                def stage_b(q, k):
                    gather_desc(k).wait()

                    @pl.loop(0, CH // 16)
                    def _(gg):
                        wv = plsc.bitcast(meta[3 * k + 2, pl.ds(gg * 16, 16)], F32)
                        for e in range(16):
                            ws = wv[e]
                            ri = gg * 16 + e
                            for cg in range(DH // 16):
                                sl2 = pl.ds(cg * 16, 16)
                                rows[k, ri, sl2] = rows[k, ri, sl2] * ws

                    scatter_desc(k).start(add=True)

                @pl.loop(0, cap // 3 + 1)
                def _(m):
                    for k in range(3):
                        q = m * 3 + k
                        with_a = q < t3

                        @pl.when(with_a & (q >= 3))
                        def _():
                            scatter_desc(k).wait()

                        @pl.when(with_a)
                        def _():
                            stage_a(q, k)

                        kb = (k - 1) % 3

                        @pl.when((q >= 1) & (q <= t3))
                        def _():
                            stage_b(q - 1, kb)

                @pl.when(t3 > 0)
                def _():
                    for k in range(3):
                        scatter_desc(k).wait()

            plsc.subcore_barrier()
            o0 = s * out_ps
            pltpu.sync_copy(acc.at[pl.ds(o0, out_ps)],
                            out.at[pl.ds(base + o0, out_ps)])
            plsc.subcore_barrier()

    return edge_k


def _pad_rows(x, n_pad):
    return jnp.pad(x, ((0, n_pad - x.shape[0]), (0, 0)))


def _pad_1d(x, e_pad):
    return jnp.pad(x, (0, e_pad - x.shape[0]))


def kernel(x_cl, x_cc, x_al, x_ac, x_pt, x_ft, es0, es1, es2, es3,
           w0, w1, w2, w3, W_cl, b_cl, W_cc, b_cc, W_ac, b_ac,
           emb_pt, emb_ft, W_mpnn, b_mpnn):
    ess = (es0, es1, es2, es3)
    ws = (w0, w1, w2, w3)

    segd = []
    for j in range(4):
        tt = T_TGTS[j]
        bucket = _make_sc_bucket(E_PADS[j], NBS[tt], R_SIZES[tt], SEGS[tt])
        oseg, ocnt = bucket(_pad_1d(ess[j][0], E_PADS[j]),
                            _pad_1d(ess[j][1], E_PADS[j]),
                            _pad_1d(ws[j], E_PADS[j]))
        nblk = oseg.shape[0] // (3 * CH)
        segd.append((oseg.reshape(nblk, 3, CH), ocnt))

    xs = [
        _enc2(_pad_rows(x_cl, N_PADS[0]), _pad_rows(x_cc, N_PADS[0]),
              W_cl, W_cc, b_cl + b_cc),
        _enc2(_pad_rows(x_al, N_PADS[1]), _pad_rows(x_ac, N_PADS[1]),
              W_ac, W_ac, 2.0 * b_ac),
        _enc_emb(_pad_rows(x_pt, N_PADS[2]), emb_pt),
        _enc_emb(_pad_rows(x_ft, N_PADS[3]), emb_ft),
    ]

    for i in range(N_LAYERS):
        new_xs = [None] * 4
        for j in range(4):
            ts, tt = T_SRCS[j], T_TGTS[j]
            msg = _msg_mm(xs[ts], W_mpnn[i, j], b_mpnn[i, j], in_relu=(i > 0))
            edge = _make_sc_edge(N_PADS[ts], N_PADS[tt], R_SIZES[tt],
                                 NBS[tt], SEGS[tt])
            eseg, cnts = segd[j]
            new_xs[tt] = edge(msg, eseg, cnts)
        xs = new_xs

    return tuple(_relu_slice(xs[t], N_SIZES[t]) for t in range(4))


# speedup vs baseline: 1.6316x; 1.6316x over previous
import dataclasses
import functools

import jax
import jax.numpy as jnp
from jax import lax
from jax.experimental import pallas as pl
from jax.experimental.pallas import tpu as pltpu
from jax.experimental.pallas import tpu_sc as plsc

F32 = jnp.float32
I32 = jnp.int32
DH = 128
NC, NS = 2, 16
NW = NC * NS
CH = 128

N_SIZES = (50000, 50000, 10000, 10000)
T_SRCS = (0, 1, 2, 3)
T_TGTS = (1, 0, 3, 2)
N_LAYERS = 5

R_BIG, R_SMALL = 12544, 5120
N_PADS = (4 * R_BIG, 4 * R_BIG, 2 * R_SMALL, 2 * R_SMALL)
R_SIZES = (R_BIG, R_BIG, R_SMALL, R_SMALL)
NBS = (4, 4, 2, 2)
E_PADS = (200704, 200704, 100352, 100352)
SEGS = (6272, 6272, 3200, 3200)


def _mm_body(x_ref, w_ref, b_ref, o_ref, *, in_relu):
    x = x_ref[...]
    if in_relu:
        x = jnp.maximum(x, 0.0)
    acc = jnp.dot(x, w_ref[...], preferred_element_type=F32) + b_ref[...]
    o_ref[...] = jnp.maximum(acc, 0.0)


def _msg_mm(x, w, b, in_relu):
    n, k = x.shape
    blk = 512
    return pl.pallas_call(
        functools.partial(_mm_body, in_relu=in_relu),
        grid=(n // blk,),
        in_specs=[
            pl.BlockSpec((blk, k), lambda i: (i, 0)),
            pl.BlockSpec((k, DH), lambda i: (0, 0)),
            pl.BlockSpec((1, DH), lambda i: (0, 0)),
        ],
        out_specs=pl.BlockSpec((blk, DH), lambda i: (i, 0)),
        out_shape=jax.ShapeDtypeStruct((n, DH), F32),
    )(x, w, b.reshape(1, DH))


def _enc2_body(xa_ref, xb_ref, wa_ref, wb_ref, b_ref, o_ref):
    acc = jnp.dot(xa_ref[...], wa_ref[...], preferred_element_type=F32)
    acc += jnp.dot(xb_ref[...], wb_ref[...], preferred_element_type=F32)
    o_ref[...] = acc + b_ref[...]


def _enc2(xa, xb, wa, wb, b):
    n, ka = xa.shape
    kb = xb.shape[1]
    blk = 512
    return pl.pallas_call(
        _enc2_body,
        grid=(n // blk,),
        in_specs=[
            pl.BlockSpec((blk, ka), lambda i: (i, 0)),
            pl.BlockSpec((blk, kb), lambda i: (i, 0)),
            pl.BlockSpec((ka, DH), lambda i: (0, 0)),
            pl.BlockSpec((kb, DH), lambda i: (0, 0)),
            pl.BlockSpec((1, DH), lambda i: (0, 0)),
        ],
        out_specs=pl.BlockSpec((blk, DH), lambda i: (i, 0)),
        out_shape=jax.ShapeDtypeStruct((n, DH), F32),
    )(xa, xb, wa, wb, b.reshape(1, DH))


def _emb_body(x_ref, e_ref, o_ref):
    o_ref[...] = x_ref[...] * e_ref[...]


def _enc_emb(x, emb):
    n = x.shape[0]
    blk = 512
    return pl.pallas_call(
        _emb_body,
        grid=(n // blk,),
        in_specs=[
            pl.BlockSpec((blk, 1), lambda i: (i, 0)),
            pl.BlockSpec((1, DH), lambda i: (0, 0)),
        ],
        out_specs=pl.BlockSpec((blk, DH), lambda i: (i, 0)),
        out_shape=jax.ShapeDtypeStruct((n, DH), F32),
    )(x, emb.reshape(1, DH))


def _relu_body(x_ref, o_ref):
    o_ref[...] = jnp.maximum(x_ref[...], 0.0)


def _relu_slice(x, n_out):
    blk = 400
    return pl.pallas_call(
        _relu_body,
        grid=(n_out // blk,),
        in_specs=[pl.BlockSpec((blk, DH), lambda i: (i, 0))],
        out_specs=pl.BlockSpec((blk, DH), lambda i: (i, 0)),
        out_shape=jax.ShapeDtypeStruct((n_out, DH), F32),
    )(x)


def _sc_mesh():
    return plsc.VectorSubcoreMesh(core_axis_name="c", subcore_axis_name="s")


def _sc_params():
    cp = pltpu.CompilerParams()
    if "needs_layout_passes" in pltpu.CompilerParams.__dataclass_fields__:
        cp = dataclasses.replace(cp, needs_layout_passes=False)
    return cp


@functools.lru_cache(maxsize=None)
def _make_sc_bucket(e_pad, nb, r, seg):
    per_w = e_pad // NW
    full_chunks = per_w // CH
    tail16 = (per_w - full_chunks * CH) // 16
    cap = seg // CH
    slot = 3 * seg

    @functools.partial(
        pl.kernel,
        out_type=(
            jax.ShapeDtypeStruct(((NW * nb * cap + 2) * 3 * CH,), I32),
            jax.ShapeDtypeStruct((NW * 16,), I32),
        ),
        mesh=_sc_mesh(),
        scratch_types=[
            pltpu.VMEM((CH,), I32),
            pltpu.VMEM((CH,), I32),
            pltpu.VMEM((CH,), F32),
            pltpu.VMEM((nb * slot,), I32),
            pltpu.VMEM((16,), I32),
        ],
        compiler_params=_sc_params(),
    )
    def bucket_k(src, tgt, w, oseg, ocnt, srcb, tgtb, wb, stg, scnt):
        c = lax.axis_index("c")
        s = lax.axis_index("s")
        wid = c * NS + s
        ebase = wid * per_w
        lanes = lax.iota(I32, 16)

        def do_chunk(e0, n16, pos):
            n = n16 * 16
            pltpu.sync_copy(src.at[pl.ds(e0, n)], srcb.at[pl.ds(0, n)])
            pltpu.sync_copy(tgt.at[pl.ds(e0, n)], tgtb.at[pl.ds(0, n)])
            pltpu.sync_copy(w.at[pl.ds(e0, n)], wb.at[pl.ds(0, n)])
            for gg in range(n16):
                sl = pl.ds(gg * 16, 16)
                tv = tgtb[sl]
                sv = srcb[sl]
                wv = plsc.bitcast(wb[sl], I32)
                bk = jnp.where(tv >= r, 1, 0)
                for m in range(2, nb):
                    bk = bk + jnp.where(tv >= m * r, 1, 0)
                p = jnp.zeros((16,), I32)
                for b in range(nb):
                    mb = bk == b
                    pc = plsc.cumsum(jnp.where(mb, 1, 0))
                    p = jnp.where(mb, b * seg + pos[b] + pc - 1, p)
                    pos = pos + jnp.where(lanes == b, pc[15], 0)
                blk = lax.shift_right_logical(p, 7)
                lane = lax.bitwise_and(p, 127)
                ps = blk * (3 * CH) + lane
                plsc.store_scatter(stg, [ps], sv)
                plsc.store_scatter(stg, [ps + CH], tv)
                plsc.store_scatter(stg, [ps + 2 * CH], wv)
            return pos

        def body(g, pos):
            return do_chunk(ebase + g * CH, CH // 16, pos)

        pos = lax.fori_loop(0, full_chunks, body, jnp.zeros((16,), I32))
        if tail16:
            pos = do_chunk(ebase + full_chunks * CH, tail16, pos)

        scnt[pl.ds(0, 16)] = pos
        for b in range(nb):
            o0 = (wid * nb + b) * slot
            pltpu.sync_copy(stg.at[pl.ds(b * slot, slot)],
                            oseg.at[pl.ds(o0, slot)])
        pltpu.sync_copy(scnt, ocnt.at[pl.ds(wid * 16, 16)])

    return bucket_k


@functools.lru_cache(maxsize=None)
def _make_sc_edge(n_tab, n_out, r, nb, seg):
    n_pass = nb // 2
    cap = seg // CH
    acc_rows = r + 16
    zone = acc_rows // NS
    out_ps = r // NS
    zsizes = [32] * (zone // 32) + ([zone % 32] if zone % 32 else [])

    @functools.partial(
        pl.kernel,
        out_type=jax.ShapeDtypeStruct((n_out, DH), F32),
        mesh=_sc_mesh(),
        scratch_types=[
            pltpu.VMEM((3, CH), I32),
            pltpu.VMEM((1, CH), I32),
            pltpu.VMEM((1, CH, DH), F32),
            pltpu.VMEM((32, DH), F32),
            pltpu.VMEM((NW * 16,), I32),
            pltpu.VMEM_SHARED((acc_rows, DH), F32),
            pltpu.SemaphoreType.DMA,
            pltpu.SemaphoreType.DMA,
            pltpu.SemaphoreType.DMA,
            pltpu.SemaphoreType.DMA,
            pltpu.SemaphoreType.DMA,
            pltpu.SemaphoreType.DMA,
        ],
        compiler_params=_sc_params(),
    )
    def edge_k(msg, eseg, cnts, out,
               meta, lclb, rows, zbuf, cntb, acc,
               gs0, gs1, gs2, ss0, ss1, ss2):
        gsems = (gs0, gs1, gs2)
        ssems = (ss0, ss1, ss2)
        c = lax.axis_index("c")
        s = lax.axis_index("s")
        lanes = lax.iota(I32, 16)
        pltpu.sync_copy(cnts, cntb)

        z16 = jnp.zeros((16,), F32)

        @pl.loop(0, 32)
        def _(rr):
            for cg in range(DH // 16):
                zbuf[rr, pl.ds(cg * 16, 16)] = z16

        def gather_desc(k):
            return pltpu.make_async_copy(
                msg.at[meta.at[3 * k]], rows.at[k], gsems[k])

        def scatter_desc(k):
            return pltpu.make_async_copy(
                rows.at[k], acc.at[lclb.at[k]], ssems[k])

        for p in range(n_pass):
            idx = 2 * p + c
            base = idx * r

            z0 = s * zone
            off = 0
            for ncopy in zsizes:
                pltpu.sync_copy(zbuf.at[pl.ds(0, ncopy)],
                                acc.at[pl.ds(z0 + off, ncopy)])
                off += ncopy
            plsc.subcore_barrier()

            for segi in range(2):
                sgm = 2 * s + segi
                cv = cntb[pl.ds(sgm * 16, 16)]
                count = jnp.sum(jnp.where(lanes == idx, cv, 0))
                nchunks = lax.div(count + (CH - 1), CH)
                t3 = lax.div(nchunks + 2, 3) * 3
                base_ci = (sgm * nb + idx) * cap

                def stage_a(q, k):
                    pltpu.sync_copy(eseg.at[base_ci + q],
                                    meta.at[pl.ds(3 * k, 3)])
                    for gg in range(CH // 16):
                        sl = pl.ds(gg * 16, 16)
                        sv = meta[3 * k, sl]
                        meta[3 * k, sl] = jnp.minimum(
                            jnp.maximum(sv, 0), n_tab - 1)
                        eid = q * CH + gg * 16 + lanes
                        lcl = meta[3 * k + 1, sl] - base
                        inr = (eid < count) & (lcl >= 0) & (lcl < r)
                        wv = plsc.bitcast(meta[3 * k + 2, sl], F32)
                        meta[3 * k + 2, sl] = plsc.bitcast(
                            jnp.where(inr, wv, 0.0), I32)
                        lclb[k, sl] = jnp.where(inr, lcl, r)
                    pltpu.sync_copy(msg.at[meta.at[3 * k]], rows.at[k])

                def stage_b(q, k):

                    @pl.loop(0, CH // 16)
                    def _(gg):
                        wv = plsc.bitcast(meta[3 * k + 2, pl.ds(gg * 16, 16)], F32)
                        for e in range(16):
                            ws = wv[e]
                            ri = gg * 16 + e
                            for cg in range(DH // 16):
                                sl2 = pl.ds(cg * 16, 16)
                                rows[k, ri, sl2] = rows[k, ri, sl2] * ws

                    pltpu.sync_copy(rows.at[k], acc.at[lclb.at[k]],
                                    add=True)

                @pl.loop(0, cap)
                def _(g):
                    @pl.when(g * CH < count)
                    def _():
                        stage_a(g, 0)
                        stage_b(g, 0)

            plsc.subcore_barrier()
            o0 = s * out_ps
            pltpu.sync_copy(acc.at[pl.ds(o0, out_ps)],
                            out.at[pl.ds(base + o0, out_ps)])
            plsc.subcore_barrier()

    return edge_k


def _pad_rows(x, n_pad):
    return jnp.pad(x, ((0, n_pad - x.shape[0]), (0, 0)))


def _pad_1d(x, e_pad):
    return jnp.pad(x, (0, e_pad - x.shape[0]))


def kernel(x_cl, x_cc, x_al, x_ac, x_pt, x_ft, es0, es1, es2, es3,
           w0, w1, w2, w3, W_cl, b_cl, W_cc, b_cc, W_ac, b_ac,
           emb_pt, emb_ft, W_mpnn, b_mpnn):
    ess = (es0, es1, es2, es3)
    ws = (w0, w1, w2, w3)

    segd = []
    for j in range(4):
        tt = T_TGTS[j]
        bucket = _make_sc_bucket(E_PADS[j], NBS[tt], R_SIZES[tt], SEGS[tt])
        oseg, ocnt = bucket(_pad_1d(ess[j][0], E_PADS[j]),
                            _pad_1d(ess[j][1], E_PADS[j]),
                            _pad_1d(ws[j], E_PADS[j]))
        nblk = oseg.shape[0] // (3 * CH)
        segd.append((oseg.reshape(nblk, 3, CH), ocnt))

    xs = [
        _enc2(_pad_rows(x_cl, N_PADS[0]), _pad_rows(x_cc, N_PADS[0]),
              W_cl, W_cc, b_cl + b_cc),
        _enc2(_pad_rows(x_al, N_PADS[1]), _pad_rows(x_ac, N_PADS[1]),
              W_ac, W_ac, 2.0 * b_ac),
        _enc_emb(_pad_rows(x_pt, N_PADS[2]), emb_pt),
        _enc_emb(_pad_rows(x_ft, N_PADS[3]), emb_ft),
    ]

    for i in range(N_LAYERS):
        new_xs = [None] * 4
        for j in range(4):
            ts, tt = T_SRCS[j], T_TGTS[j]
            msg = _msg_mm(xs[ts], W_mpnn[i, j], b_mpnn[i, j], in_relu=(i > 0))
            edge = _make_sc_edge(N_PADS[ts], N_PADS[tt], R_SIZES[tt],
                                 NBS[tt], SEGS[tt])
            eseg, cnts = segd[j]
            new_xs[tt] = edge(msg, eseg, cnts)
        xs = new_xs

    return tuple(_relu_slice(xs[t], N_SIZES[t]) for t in range(4))


# speedup vs baseline: 1.6992x; 1.0414x over previous
import dataclasses
import functools

import jax
import jax.numpy as jnp
from jax import lax
from jax.experimental import pallas as pl
from jax.experimental.pallas import tpu as pltpu
from jax.experimental.pallas import tpu_sc as plsc

F32 = jnp.float32
I32 = jnp.int32
DH = 128
NC, NS = 2, 16
NW = NC * NS
CH = 128

N_SIZES = (50000, 50000, 10000, 10000)
T_SRCS = (0, 1, 2, 3)
T_TGTS = (1, 0, 3, 2)
N_LAYERS = 5

R_BIG, R_SMALL = 12544, 5120
N_PADS = (4 * R_BIG, 4 * R_BIG, 2 * R_SMALL, 2 * R_SMALL)
R_SIZES = (R_BIG, R_BIG, R_SMALL, R_SMALL)
NBS = (4, 4, 2, 2)
E_PADS = (200704, 200704, 100352, 100352)
SEGS = (6272, 6272, 3200, 3200)


def _mm_body(x_ref, w_ref, b_ref, o_ref, *, in_relu):
    x = x_ref[...]
    if in_relu:
        x = jnp.maximum(x, 0.0)
    acc = jnp.dot(x, w_ref[...], preferred_element_type=F32) + b_ref[...]
    o_ref[...] = jnp.maximum(acc, 0.0)


def _msg_mm(x, w, b, in_relu):
    n, k = x.shape
    blk = 512
    return pl.pallas_call(
        functools.partial(_mm_body, in_relu=in_relu),
        grid=(n // blk,),
        in_specs=[
            pl.BlockSpec((blk, k), lambda i: (i, 0)),
            pl.BlockSpec((k, DH), lambda i: (0, 0)),
            pl.BlockSpec((1, DH), lambda i: (0, 0)),
        ],
        out_specs=pl.BlockSpec((blk, DH), lambda i: (i, 0)),
        out_shape=jax.ShapeDtypeStruct((n, DH), F32),
    )(x, w, b.reshape(1, DH))


def _enc2_body(xa_ref, xb_ref, wa_ref, wb_ref, b_ref, o_ref):
    acc = jnp.dot(xa_ref[...], wa_ref[...], preferred_element_type=F32)
    acc += jnp.dot(xb_ref[...], wb_ref[...], preferred_element_type=F32)
    o_ref[...] = acc + b_ref[...]


def _enc2(xa, xb, wa, wb, b):
    n, ka = xa.shape
    kb = xb.shape[1]
    blk = 512
    return pl.pallas_call(
        _enc2_body,
        grid=(n // blk,),
        in_specs=[
            pl.BlockSpec((blk, ka), lambda i: (i, 0)),
            pl.BlockSpec((blk, kb), lambda i: (i, 0)),
            pl.BlockSpec((ka, DH), lambda i: (0, 0)),
            pl.BlockSpec((kb, DH), lambda i: (0, 0)),
            pl.BlockSpec((1, DH), lambda i: (0, 0)),
        ],
        out_specs=pl.BlockSpec((blk, DH), lambda i: (i, 0)),
        out_shape=jax.ShapeDtypeStruct((n, DH), F32),
    )(xa, xb, wa, wb, b.reshape(1, DH))


def _emb_body(x_ref, e_ref, o_ref):
    o_ref[...] = x_ref[...] * e_ref[...]


def _enc_emb(x, emb):
    n = x.shape[0]
    blk = 512
    return pl.pallas_call(
        _emb_body,
        grid=(n // blk,),
        in_specs=[
            pl.BlockSpec((blk, 1), lambda i: (i, 0)),
            pl.BlockSpec((1, DH), lambda i: (0, 0)),
        ],
        out_specs=pl.BlockSpec((blk, DH), lambda i: (i, 0)),
        out_shape=jax.ShapeDtypeStruct((n, DH), F32),
    )(x, emb.reshape(1, DH))


def _relu_body(x_ref, o_ref):
    o_ref[...] = jnp.maximum(x_ref[...], 0.0)


def _relu_slice(x, n_out):
    blk = 400
    return pl.pallas_call(
        _relu_body,
        grid=(n_out // blk,),
        in_specs=[pl.BlockSpec((blk, DH), lambda i: (i, 0))],
        out_specs=pl.BlockSpec((blk, DH), lambda i: (i, 0)),
        out_shape=jax.ShapeDtypeStruct((n_out, DH), F32),
    )(x)


def _sc_mesh():
    return plsc.VectorSubcoreMesh(core_axis_name="c", subcore_axis_name="s")


def _sc_params():
    cp = pltpu.CompilerParams()
    if "needs_layout_passes" in pltpu.CompilerParams.__dataclass_fields__:
        cp = dataclasses.replace(cp, needs_layout_passes=False)
    return cp


@functools.lru_cache(maxsize=None)
def _make_sc_bucket(e_pad, nb, r, seg):
    per_w = e_pad // NW
    full_chunks = per_w // CH
    tail16 = (per_w - full_chunks * CH) // 16
    cap = seg // CH
    slot = 3 * seg

    @functools.partial(
        pl.kernel,
        out_type=(
            jax.ShapeDtypeStruct(((NW * nb * cap + 2) * 3 * CH,), I32),
            jax.ShapeDtypeStruct((NW * 16,), I32),
        ),
        mesh=_sc_mesh(),
        scratch_types=[
            pltpu.VMEM((CH,), I32),
            pltpu.VMEM((CH,), I32),
            pltpu.VMEM((CH,), F32),
            pltpu.VMEM((nb * slot,), I32),
            pltpu.VMEM((16,), I32),
        ],
        compiler_params=_sc_params(),
    )
    def bucket_k(src, tgt, w, oseg, ocnt, srcb, tgtb, wb, stg, scnt):
        c = lax.axis_index("c")
        s = lax.axis_index("s")
        wid = c * NS + s
        ebase = wid * per_w
        lanes = lax.iota(I32, 16)

        def do_chunk(e0, n16, pos):
            n = n16 * 16
            pltpu.sync_copy(src.at[pl.ds(e0, n)], srcb.at[pl.ds(0, n)])
            pltpu.sync_copy(tgt.at[pl.ds(e0, n)], tgtb.at[pl.ds(0, n)])
            pltpu.sync_copy(w.at[pl.ds(e0, n)], wb.at[pl.ds(0, n)])
            for gg in range(n16):
                sl = pl.ds(gg * 16, 16)
                tv = tgtb[sl]
                sv = srcb[sl]
                wv = plsc.bitcast(wb[sl], I32)
                bk = jnp.where(tv >= r, 1, 0)
                for m in range(2, nb):
                    bk = bk + jnp.where(tv >= m * r, 1, 0)
                p = jnp.zeros((16,), I32)
                for b in range(nb):
                    mb = bk == b
                    pc = plsc.cumsum(jnp.where(mb, 1, 0))
                    p = jnp.where(mb, b * seg + pos[b] + pc - 1, p)
                    pos = pos + jnp.where(lanes == b, pc[15], 0)
                blk = lax.shift_right_logical(p, 7)
                lane = lax.bitwise_and(p, 127)
                ps = blk * (3 * CH) + lane
                plsc.store_scatter(stg, [ps], sv)
                plsc.store_scatter(stg, [ps + CH], tv)
                plsc.store_scatter(stg, [ps + 2 * CH], wv)
            return pos

        def body(g, pos):
            return do_chunk(ebase + g * CH, CH // 16, pos)

        pos = lax.fori_loop(0, full_chunks, body, jnp.zeros((16,), I32))
        if tail16:
            pos = do_chunk(ebase + full_chunks * CH, tail16, pos)

        scnt[pl.ds(0, 16)] = pos
        for b in range(nb):
            o0 = (wid * nb + b) * slot
            pltpu.sync_copy(stg.at[pl.ds(b * slot, slot)],
                            oseg.at[pl.ds(o0, slot)])
        pltpu.sync_copy(scnt, ocnt.at[pl.ds(wid * 16, 16)])

    return bucket_k


@functools.lru_cache(maxsize=None)
def _make_sc_edge(n_tab, n_out, r, nb, seg):
    n_pass = nb // 2
    cap = seg // CH
    acc_rows = r + 16
    zone = acc_rows // NS
    out_ps = r // NS
    zsizes = [32] * (zone // 32) + ([zone % 32] if zone % 32 else [])

    @functools.partial(
        pl.kernel,
        out_type=jax.ShapeDtypeStruct((n_out, DH), F32),
        mesh=_sc_mesh(),
        scratch_types=[
            pltpu.VMEM((6, CH), I32),
            pltpu.VMEM((1, CH), I32),
            pltpu.VMEM((1, CH, DH), F32),
            pltpu.VMEM((32, DH), F32),
            pltpu.VMEM((NW * 16,), I32),
            pltpu.VMEM_SHARED((acc_rows, DH), F32),
            pltpu.SemaphoreType.DMA,
            pltpu.SemaphoreType.DMA,
            pltpu.SemaphoreType.DMA,
            pltpu.SemaphoreType.DMA,
            pltpu.SemaphoreType.DMA,
            pltpu.SemaphoreType.DMA,
        ],
        compiler_params=_sc_params(),
    )
    def edge_k(msg, eseg, cnts, out,
               meta, lclb, rows, zbuf, cntb, acc,
               gs0, gs1, gs2, ss0, ss1, ss2):
        gsems = (gs0, gs1, gs2)
        ssems = (ss0, ss1, ss2)
        c = lax.axis_index("c")
        s = lax.axis_index("s")
        lanes = lax.iota(I32, 16)
        pltpu.sync_copy(cnts, cntb)

        z16 = jnp.zeros((16,), F32)

        @pl.loop(0, 32)
        def _(rr):
            for cg in range(DH // 16):
                zbuf[rr, pl.ds(cg * 16, 16)] = z16

        def gather_desc(k):
            return pltpu.make_async_copy(
                msg.at[meta.at[3 * k]], rows.at[k], gsems[k])

        def scatter_desc(k):
            return pltpu.make_async_copy(
                rows.at[k], acc.at[lclb.at[k]], ssems[k])

        for p in range(n_pass):
            idx = 2 * p + c
            base = idx * r

            z0 = s * zone
            off = 0
            for ncopy in zsizes:
                pltpu.sync_copy(zbuf.at[pl.ds(0, ncopy)],
                                acc.at[pl.ds(z0 + off, ncopy)])
                off += ncopy
            plsc.subcore_barrier()

            for segi in range(2):
                sgm = 2 * s + segi
                cv = cntb[pl.ds(sgm * 16, 16)]
                count = jnp.sum(jnp.where(lanes == idx, cv, 0))
                nchunks = lax.div(count + (CH - 1), CH)
                t3 = lax.div(nchunks + 2, 3) * 3
                base_ci = (sgm * nb + idx) * cap

                def meta_desc(q, k):
                    return pltpu.make_async_copy(
                        eseg.at[base_ci + q], meta.at[pl.ds(3 * k, 3)],
                        gsems[k])

                def stage_a(q, k):
                    meta_desc(q, k).wait()

                    @pl.when(q + 1 < nchunks)
                    def _():
                        meta_desc(q + 1, 1 - k).start()

                    for gg in range(CH // 16):
                        sl = pl.ds(gg * 16, 16)
                        sv = meta[3 * k, sl]
                        meta[3 * k, sl] = jnp.minimum(
                            jnp.maximum(sv, 0), n_tab - 1)
                        eid = q * CH + gg * 16 + lanes
                        lcl = meta[3 * k + 1, sl] - base
                        inr = (eid < count) & (lcl >= 0) & (lcl < r)
                        wv = plsc.bitcast(meta[3 * k + 2, sl], F32)
                        meta[3 * k + 2, sl] = plsc.bitcast(
                            jnp.where(inr, wv, 0.0), I32)
                        lclb[0, sl] = jnp.where(inr, lcl, r)
                    pltpu.sync_copy(msg.at[meta.at[3 * k]], rows.at[0])

                def stage_b(q, k):

                    @pl.loop(0, CH // 16)
                    def _(gg):
                        wv = plsc.bitcast(meta[3 * k + 2, pl.ds(gg * 16, 16)], F32)
                        for e in range(16):
                            ws = wv[e]
                            ri = gg * 16 + e
                            for cg in range(DH // 16):
                                sl2 = pl.ds(cg * 16, 16)
                                rows[0, ri, sl2] = rows[0, ri, sl2] * ws

                    pltpu.sync_copy(rows.at[0], acc.at[lclb.at[0]],
                                    add=True)

                @pl.when(nchunks > 0)
                def _():
                    meta_desc(0, 0).start()

                @pl.loop(0, cap // 2 + 1)
                def _(m):
                    for k in range(2):
                        g = m * 2 + k

                        @pl.when(g < nchunks)
                        def _():
                            stage_a(g, k)
                            stage_b(g, k)

            plsc.subcore_barrier()
            o0 = s * out_ps
            pltpu.sync_copy(acc.at[pl.ds(o0, out_ps)],
                            out.at[pl.ds(base + o0, out_ps)])
            plsc.subcore_barrier()

    return edge_k


def _pad_rows(x, n_pad):
    return jnp.pad(x, ((0, n_pad - x.shape[0]), (0, 0)))


def _pad_1d(x, e_pad):
    return jnp.pad(x, (0, e_pad - x.shape[0]))


def kernel(x_cl, x_cc, x_al, x_ac, x_pt, x_ft, es0, es1, es2, es3,
           w0, w1, w2, w3, W_cl, b_cl, W_cc, b_cc, W_ac, b_ac,
           emb_pt, emb_ft, W_mpnn, b_mpnn):
    ess = (es0, es1, es2, es3)
    ws = (w0, w1, w2, w3)

    segd = []
    for j in range(4):
        tt = T_TGTS[j]
        bucket = _make_sc_bucket(E_PADS[j], NBS[tt], R_SIZES[tt], SEGS[tt])
        oseg, ocnt = bucket(_pad_1d(ess[j][0], E_PADS[j]),
                            _pad_1d(ess[j][1], E_PADS[j]),
                            _pad_1d(ws[j], E_PADS[j]))
        nblk = oseg.shape[0] // (3 * CH)
        segd.append((oseg.reshape(nblk, 3, CH), ocnt))

    xs = [
        _enc2(_pad_rows(x_cl, N_PADS[0]), _pad_rows(x_cc, N_PADS[0]),
              W_cl, W_cc, b_cl + b_cc),
        _enc2(_pad_rows(x_al, N_PADS[1]), _pad_rows(x_ac, N_PADS[1]),
              W_ac, W_ac, 2.0 * b_ac),
        _enc_emb(_pad_rows(x_pt, N_PADS[2]), emb_pt),
        _enc_emb(_pad_rows(x_ft, N_PADS[3]), emb_ft),
    ]

    for i in range(N_LAYERS):
        new_xs = [None] * 4
        for j in range(4):
            ts, tt = T_SRCS[j], T_TGTS[j]
            msg = _msg_mm(xs[ts], W_mpnn[i, j], b_mpnn[i, j], in_relu=(i > 0))
            edge = _make_sc_edge(N_PADS[ts], N_PADS[tt], R_SIZES[tt],
                                 NBS[tt], SEGS[tt])
            eseg, cnts = segd[j]
            new_xs[tt] = edge(msg, eseg, cnts)
        xs = new_xs

    return tuple(_relu_slice(xs[t], N_SIZES[t]) for t in range(4))


# speedup vs baseline: 1.7032x; 1.0024x over previous
import dataclasses
import functools

import jax
import jax.numpy as jnp
from jax import lax
from jax.experimental import pallas as pl
from jax.experimental.pallas import tpu as pltpu
from jax.experimental.pallas import tpu_sc as plsc

F32 = jnp.float32
I32 = jnp.int32
DH = 128
NC, NS = 2, 16
NW = NC * NS
CH = 128

N_SIZES = (50000, 50000, 10000, 10000)
T_SRCS = (0, 1, 2, 3)
T_TGTS = (1, 0, 3, 2)
N_LAYERS = 5

R_BIG, R_SMALL = 12544, 5120
N_PADS = (4 * R_BIG, 4 * R_BIG, 2 * R_SMALL, 2 * R_SMALL)
R_SIZES = (R_BIG, R_BIG, R_SMALL, R_SMALL)
NBS = (4, 4, 2, 2)
E_PADS = (200704, 200704, 100352, 100352)
SEGS = (6272, 6272, 3200, 3200)


def _mm_body(x_ref, w_ref, b_ref, o_ref, *, in_relu):
    x = x_ref[...]
    if in_relu:
        x = jnp.maximum(x, 0.0)
    acc = jnp.dot(x, w_ref[...], preferred_element_type=F32) + b_ref[...]
    o_ref[...] = jnp.maximum(acc, 0.0)


def _msg_mm(x, w, b, in_relu):
    n, k = x.shape
    blk = 512
    return pl.pallas_call(
        functools.partial(_mm_body, in_relu=in_relu),
        grid=(n // blk,),
        in_specs=[
            pl.BlockSpec((blk, k), lambda i: (i, 0)),
            pl.BlockSpec((k, DH), lambda i: (0, 0)),
            pl.BlockSpec((1, DH), lambda i: (0, 0)),
        ],
        out_specs=pl.BlockSpec((blk, DH), lambda i: (i, 0)),
        out_shape=jax.ShapeDtypeStruct((n, DH), F32),
    )(x, w, b.reshape(1, DH))


def _enc2_body(xa_ref, xb_ref, wa_ref, wb_ref, b_ref, o_ref):
    acc = jnp.dot(xa_ref[...], wa_ref[...], preferred_element_type=F32)
    acc += jnp.dot(xb_ref[...], wb_ref[...], preferred_element_type=F32)
    o_ref[...] = acc + b_ref[...]


def _enc2(xa, xb, wa, wb, b):
    n, ka = xa.shape
    kb = xb.shape[1]
    blk = 512
    return pl.pallas_call(
        _enc2_body,
        grid=(n // blk,),
        in_specs=[
            pl.BlockSpec((blk, ka), lambda i: (i, 0)),
            pl.BlockSpec((blk, kb), lambda i: (i, 0)),
            pl.BlockSpec((ka, DH), lambda i: (0, 0)),
            pl.BlockSpec((kb, DH), lambda i: (0, 0)),
            pl.BlockSpec((1, DH), lambda i: (0, 0)),
        ],
        out_specs=pl.BlockSpec((blk, DH), lambda i: (i, 0)),
        out_shape=jax.ShapeDtypeStruct((n, DH), F32),
    )(xa, xb, wa, wb, b.reshape(1, DH))


def _emb_body(x_ref, e_ref, o_ref):
    o_ref[...] = x_ref[...] * e_ref[...]


def _enc_emb(x, emb):
    n = x.shape[0]
    blk = 512
    return pl.pallas_call(
        _emb_body,
        grid=(n // blk,),
        in_specs=[
            pl.BlockSpec((blk, 1), lambda i: (i, 0)),
            pl.BlockSpec((1, DH), lambda i: (0, 0)),
        ],
        out_specs=pl.BlockSpec((blk, DH), lambda i: (i, 0)),
        out_shape=jax.ShapeDtypeStruct((n, DH), F32),
    )(x, emb.reshape(1, DH))


def _relu_body(x_ref, o_ref):
    o_ref[...] = jnp.maximum(x_ref[...], 0.0)


def _relu_slice(x, n_out):
    blk = 400
    return pl.pallas_call(
        _relu_body,
        grid=(n_out // blk,),
        in_specs=[pl.BlockSpec((blk, DH), lambda i: (i, 0))],
        out_specs=pl.BlockSpec((blk, DH), lambda i: (i, 0)),
        out_shape=jax.ShapeDtypeStruct((n_out, DH), F32),
    )(x)


def _sc_mesh():
    return plsc.VectorSubcoreMesh(core_axis_name="c", subcore_axis_name="s")


def _sc_params():
    cp = pltpu.CompilerParams()
    if "needs_layout_passes" in pltpu.CompilerParams.__dataclass_fields__:
        cp = dataclasses.replace(cp, needs_layout_passes=False)
    return cp


@functools.lru_cache(maxsize=None)
def _make_sc_bucket(e_pad, nb, r, seg):
    per_w = e_pad // NW
    full_chunks = per_w // CH
    tail16 = (per_w - full_chunks * CH) // 16
    cap = seg // CH
    slot = 3 * seg

    @functools.partial(
        pl.kernel,
        out_type=(
            jax.ShapeDtypeStruct(((NW * nb * cap + 2) * 3 * CH,), I32),
            jax.ShapeDtypeStruct((NW * 16,), I32),
        ),
        mesh=_sc_mesh(),
        scratch_types=[
            pltpu.VMEM((CH,), I32),
            pltpu.VMEM((CH,), I32),
            pltpu.VMEM((CH,), F32),
            pltpu.VMEM((nb * slot,), I32),
            pltpu.VMEM((16,), I32),
        ],
        compiler_params=_sc_params(),
    )
    def bucket_k(src, tgt, w, oseg, ocnt, srcb, tgtb, wb, stg, scnt):
        c = lax.axis_index("c")
        s = lax.axis_index("s")
        wid = c * NS + s
        ebase = wid * per_w
        lanes = lax.iota(I32, 16)

        def do_chunk(e0, n16, pos):
            n = n16 * 16
            pltpu.sync_copy(src.at[pl.ds(e0, n)], srcb.at[pl.ds(0, n)])
            pltpu.sync_copy(tgt.at[pl.ds(e0, n)], tgtb.at[pl.ds(0, n)])
            pltpu.sync_copy(w.at[pl.ds(e0, n)], wb.at[pl.ds(0, n)])
            for gg in range(n16):
                sl = pl.ds(gg * 16, 16)
                tv = tgtb[sl]
                sv = srcb[sl]
                wv = plsc.bitcast(wb[sl], I32)
                bk = jnp.where(tv >= r, 1, 0)
                for m in range(2, nb):
                    bk = bk + jnp.where(tv >= m * r, 1, 0)
                p = jnp.zeros((16,), I32)
                for b in range(nb):
                    mb = bk == b
                    pc = plsc.cumsum(jnp.where(mb, 1, 0))
                    p = jnp.where(mb, b * seg + pos[b] + pc - 1, p)
                    pos = pos + jnp.where(lanes == b, pc[15], 0)
                blk = lax.shift_right_logical(p, 7)
                lane = lax.bitwise_and(p, 127)
                ps = blk * (3 * CH) + lane
                plsc.store_scatter(stg, [ps], sv)
                plsc.store_scatter(stg, [ps + CH], tv)
                plsc.store_scatter(stg, [ps + 2 * CH], wv)
            return pos

        def body(g, pos):
            return do_chunk(ebase + g * CH, CH // 16, pos)

        pos = lax.fori_loop(0, full_chunks, body, jnp.zeros((16,), I32))
        if tail16:
            pos = do_chunk(ebase + full_chunks * CH, tail16, pos)

        scnt[pl.ds(0, 16)] = pos
        for b in range(nb):
            o0 = (wid * nb + b) * slot
            pltpu.sync_copy(stg.at[pl.ds(b * slot, slot)],
                            oseg.at[pl.ds(o0, slot)])
        pltpu.sync_copy(scnt, ocnt.at[pl.ds(wid * 16, 16)])

    return bucket_k


@functools.lru_cache(maxsize=None)
def _make_sc_edge(n_tab, n_out, r, nb, seg):
    n_pass = nb // 2
    cap = seg // CH
    acc_rows = r + 16
    zone = acc_rows // NS
    out_ps = r // NS
    zsizes = [32] * (zone // 32) + ([zone % 32] if zone % 32 else [])

    @functools.partial(
        pl.kernel,
        out_type=jax.ShapeDtypeStruct((n_out, DH), F32),
        mesh=_sc_mesh(),
        scratch_types=[
            pltpu.VMEM((6, CH), I32),
            pltpu.VMEM((1, CH), I32),
            pltpu.VMEM((1, CH, DH), F32),
            pltpu.VMEM((32, DH), F32),
            pltpu.VMEM((NW * 16,), I32),
            pltpu.VMEM_SHARED((acc_rows, DH), F32),
            pltpu.SemaphoreType.DMA,
            pltpu.SemaphoreType.DMA,
        ],
        compiler_params=_sc_params(),
    )
    def edge_k(msg, eseg, cnts, out,
               meta, lclb, rows, zbuf, cntb, acc, ms0, ms1):
        msems = (ms0, ms1)
        c = lax.axis_index("c")
        s = lax.axis_index("s")
        lanes = lax.iota(I32, 16)
        pltpu.sync_copy(cnts, cntb)

        z16 = jnp.zeros((16,), F32)

        @pl.loop(0, 32)
        def _(rr):
            for cg in range(DH // 16):
                zbuf[rr, pl.ds(cg * 16, 16)] = z16

        for p in range(n_pass):
            idx = 2 * p + c
            base = idx * r

            z0 = s * zone
            off = 0
            for ncopy in zsizes:
                pltpu.sync_copy(zbuf.at[pl.ds(0, ncopy)],
                                acc.at[pl.ds(z0 + off, ncopy)])
                off += ncopy
            plsc.subcore_barrier()

            for segi in range(2):
                sgm = 2 * s + segi
                cv = cntb[pl.ds(sgm * 16, 16)]
                count = jnp.sum(jnp.where(lanes == idx, cv, 0))
                nchunks = lax.div(count + (CH - 1), CH)
                base_ci = (sgm * nb + idx) * cap

                def meta_desc(q, k):
                    return pltpu.make_async_copy(
                        eseg.at[base_ci + q], meta.at[pl.ds(3 * k, 3)],
                        msems[k])

                def stage_a(q, k):
                    meta_desc(q, k).wait()

                    @pl.when(q + 1 < nchunks)
                    def _():
                        meta_desc(q + 1, 1 - k).start()

                    for gg in range(CH // 16):
                        sl = pl.ds(gg * 16, 16)
                        sv = meta[3 * k, sl]
                        meta[3 * k, sl] = jnp.minimum(
                            jnp.maximum(sv, 0), n_tab - 1)
                        eid = q * CH + gg * 16 + lanes
                        lcl = meta[3 * k + 1, sl] - base
                        inr = (eid < count) & (lcl >= 0) & (lcl < r)
                        wv = plsc.bitcast(meta[3 * k + 2, sl], F32)
                        meta[3 * k + 2, sl] = plsc.bitcast(
                            jnp.where(inr, wv, 0.0), I32)
                        lclb[0, sl] = jnp.where(inr, lcl, r)
                    pltpu.sync_copy(msg.at[meta.at[3 * k]], rows.at[0])

                def stage_b(q, k):

                    @pl.loop(0, CH // 16)
                    def _(gg):
                        wv = plsc.bitcast(meta[3 * k + 2, pl.ds(gg * 16, 16)], F32)
                        for e in range(16):
                            ws = wv[e]
                            ri = gg * 16 + e
                            for cg in range(DH // 16):
                                sl2 = pl.ds(cg * 16, 16)
                                rows[0, ri, sl2] = rows[0, ri, sl2] * ws

                    pltpu.sync_copy(rows.at[0], acc.at[lclb.at[0]],
                                    add=True)

                @pl.when(nchunks > 0)
                def _():
                    meta_desc(0, 0).start()

                @pl.loop(0, cap // 2 + 1)
                def _(m):
                    for k in range(2):
                        g = m * 2 + k

                        @pl.when(g < nchunks)
                        def _():
                            stage_a(g, k)
                            stage_b(g, k)

            plsc.subcore_barrier()
            o0 = s * out_ps
            pltpu.sync_copy(acc.at[pl.ds(o0, out_ps)],
                            out.at[pl.ds(base + o0, out_ps)])
            plsc.subcore_barrier()

    return edge_k


def _pad_rows(x, n_pad):
    return jnp.pad(x, ((0, n_pad - x.shape[0]), (0, 0)))


def _pad_1d(x, e_pad):
    return jnp.pad(x, (0, e_pad - x.shape[0]))


def kernel(x_cl, x_cc, x_al, x_ac, x_pt, x_ft, es0, es1, es2, es3,
           w0, w1, w2, w3, W_cl, b_cl, W_cc, b_cc, W_ac, b_ac,
           emb_pt, emb_ft, W_mpnn, b_mpnn):
    ess = (es0, es1, es2, es3)
    ws = (w0, w1, w2, w3)

    segd = []
    for j in range(4):
        tt = T_TGTS[j]
        bucket = _make_sc_bucket(E_PADS[j], NBS[tt], R_SIZES[tt], SEGS[tt])
        oseg, ocnt = bucket(_pad_1d(ess[j][0], E_PADS[j]),
                            _pad_1d(ess[j][1], E_PADS[j]),
                            _pad_1d(ws[j], E_PADS[j]))
        nblk = oseg.shape[0] // (3 * CH)
        segd.append((oseg.reshape(nblk, 3, CH), ocnt))

    xs = [
        _enc2(_pad_rows(x_cl, N_PADS[0]), _pad_rows(x_cc, N_PADS[0]),
              W_cl, W_cc, b_cl + b_cc),
        _enc2(_pad_rows(x_al, N_PADS[1]), _pad_rows(x_ac, N_PADS[1]),
              W_ac, W_ac, 2.0 * b_ac),
        _enc_emb(_pad_rows(x_pt, N_PADS[2]), emb_pt),
        _enc_emb(_pad_rows(x_ft, N_PADS[3]), emb_ft),
    ]

    for i in range(N_LAYERS):
        new_xs = [None] * 4
        for j in range(4):
            ts, tt = T_SRCS[j], T_TGTS[j]
            msg = _msg_mm(xs[ts], W_mpnn[i, j], b_mpnn[i, j], in_relu=(i > 0))
            edge = _make_sc_edge(N_PADS[ts], N_PADS[tt], R_SIZES[tt],
                                 NBS[tt], SEGS[tt])
            eseg, cnts = segd[j]
            new_xs[tt] = edge(msg, eseg, cnts)
        xs = new_xs

    return tuple(_relu_slice(xs[t], N_SIZES[t]) for t in range(4))
